# baseline (speedup 1.0000x reference)
import functools
import math

import jax
import jax.numpy as jnp
from jax import lax
from jax.experimental import pallas as pl
from jax.experimental.pallas import tpu as pltpu
from jax.experimental.pallas import tpu_sc as plsc

_EMB = 1024
_SEQ = 2048
_BATCH = 4
_ROWS = _SEQ * _BATCH
_NC, _NS = 2, 16
_NW = _NC * _NS
_RPW = _ROWS // _NW
_C = 32
_NCHUNK = _RPW // _C
_LANES = 16
_SCALE = math.sqrt(_EMB)


def _sc_embed(tok_flat, table, pe):
    mesh = plsc.VectorSubcoreMesh(core_axis_name="c", subcore_axis_name="s")

    @functools.partial(
        pl.kernel,
        out_type=jax.ShapeDtypeStruct((_ROWS, _EMB), jnp.float32),
        mesh=mesh,
        scratch_types=[
            pltpu.VMEM((_C,), jnp.int32),
            pltpu.VMEM((_C, _EMB), jnp.float32),
            pltpu.VMEM((_C // _BATCH, _EMB), jnp.float32),
            pltpu.SemaphoreType.DMA,
        ],
    )
    def k(tok_hbm, table_hbm, pe_hbm, out_hbm, idx_v, rows_v, pos_v, sem):
        wid = lax.axis_index("s") * _NC + lax.axis_index("c")
        base = wid * _RPW

        def chunk(g, carry):
            off = pl.multiple_of(base + g * _C, _C)
            pltpu.sync_copy(tok_hbm.at[pl.ds(off, _C)], idx_v)
            gat = pltpu.async_copy(table_hbm.at[idx_v], rows_v, sem)
            poff = pl.multiple_of(off // _BATCH, _C // _BATCH)
            pltpu.sync_copy(pe_hbm.at[pl.ds(poff, _C // _BATCH)], pos_v)
            gat.wait()

            def row(r, c2):
                pr = r // _BATCH
                for j in range(_EMB // _LANES):
                    sl = pl.ds(j * _LANES, _LANES)
                    rows_v[r, sl] = rows_v[r, sl] * _SCALE + pos_v[pr, sl]
                return c2

            lax.fori_loop(0, _C, row, 0)
            pltpu.sync_copy(rows_v, out_hbm.at[pl.ds(off, _C)])
            return carry

        lax.fori_loop(0, _NCHUNK, chunk, 0)

    return k(tok_flat, table, pe)


def kernel(tokens, table, pos_embedding):
    tok_flat = tokens.reshape(-1).astype(jnp.int32)
    pe = pos_embedding[:_SEQ, 0, :]
    out = _sc_embed(tok_flat, table, pe)
    return out.reshape(_SEQ, _BATCH, _EMB)


# speedup vs baseline: 1.6407x; 1.6407x over previous
import functools
import math

import jax
import jax.numpy as jnp
from jax import lax
from jax.experimental import pallas as pl
from jax.experimental.pallas import tpu as pltpu
from jax.experimental.pallas import tpu_sc as plsc

_EMB = 1024
_SEQ = 2048
_BATCH = 4
_ROWS = _SEQ * _BATCH
_NC, _NS = 2, 16
_NW = _NC * _NS
_RPW = _ROWS // _NW
_C = 32
_NCHUNK = _RPW // _C
_PC = _C // _BATCH
_LANES = 16
_SCALE = math.sqrt(_EMB)


def _sc_embed(tok_flat, table, pe):
    mesh = plsc.VectorSubcoreMesh(core_axis_name="c", subcore_axis_name="s")

    @functools.partial(
        pl.kernel,
        out_type=jax.ShapeDtypeStruct((_ROWS, _EMB), jnp.float32),
        mesh=mesh,
        scratch_types=[
            pltpu.VMEM((_RPW,), jnp.int32),
            pltpu.VMEM((2, _C, _EMB), jnp.float32),
            pltpu.VMEM((2, _PC, _EMB), jnp.float32),
            pltpu.SemaphoreType.DMA,
            pltpu.SemaphoreType.DMA,
            pltpu.SemaphoreType.DMA,
        ],
    )
    def k(tok_hbm, table_hbm, pe_hbm, out_hbm, idx_v, rows2, pos2, gsem, psem, osem):
        wid = lax.axis_index("s") * _NC + lax.axis_index("c")
        base = wid * _RPW
        pltpu.sync_copy(tok_hbm.at[pl.ds(pl.multiple_of(base, _RPW), _RPW)], idx_v)

        def issue(g, slot):
            ioff = pl.multiple_of(g * _C, _C)
            off = pl.multiple_of(base + g * _C, _C)
            pltpu.async_copy(
                table_hbm.at[idx_v.at[pl.ds(ioff, _C)]], rows2.at[slot], gsem)
            poff = pl.multiple_of(off // _BATCH, _PC)
            pltpu.async_copy(pe_hbm.at[pl.ds(poff, _PC)], pos2.at[slot], psem)

        issue(0, 0)

        def wait_out():
            pltpu.make_async_copy(
                rows2.at[0], out_hbm.at[pl.ds(0, _C)], osem).wait()

        def chunk(g, carry):
            b = lax.rem(g, 2)
            nxt = 1 - b

            @pl.when(g + 1 < _NCHUNK)
            def _prefetch():
                @pl.when(g >= 1)
                def _drain_prev_out():
                    wait_out()
                issue(g + 1, nxt)

            pltpu.make_async_copy(
                table_hbm.at[idx_v.at[pl.ds(0, _C)]], rows2.at[b], gsem).wait()
            pltpu.make_async_copy(
                pe_hbm.at[pl.ds(0, _PC)], pos2.at[b], psem).wait()

            def quad(q, c2):
                for j in range(_EMB // _LANES):
                    sl = pl.ds(j * _LANES, _LANES)
                    pv = pos2[b, q, sl]
                    for t in range(_BATCH):
                        r = q * _BATCH + t
                        rows2[b, r, sl] = rows2[b, r, sl] * _SCALE + pv
                return c2

            lax.fori_loop(0, _PC, quad, 0)
            off = pl.multiple_of(base + g * _C, _C)
            pltpu.async_copy(rows2.at[b], out_hbm.at[pl.ds(off, _C)], osem)
            return carry

        lax.fori_loop(0, _NCHUNK, chunk, 0)
        wait_out()
        wait_out()

    return k(tok_flat, table, pe)


def kernel(tokens, table, pos_embedding):
    tok_flat = tokens.reshape(-1).astype(jnp.int32)
    pe = pos_embedding[:_SEQ, 0, :]
    out = _sc_embed(tok_flat, table, pe)
    return out.reshape(_SEQ, _BATCH, _EMB)
